# Initial kernel scaffold; baseline (speedup 1.0000x reference)
import jax
import jax.numpy as jnp
from jax.experimental import pallas as pl


def kernel(features, edges, weights, W1, b1, W2, b2, W3, b3, W4, b4):
    raise NotImplementedError("write your pallas kernel here")


# speedup vs baseline: 19.4188x; 19.4188x over previous
import functools

import jax
import jax.numpy as jnp
from jax import lax
from jax.experimental import pallas as pl
from jax.experimental.pallas import tpu as pltpu, tpu_sc as plsc

NC, NS, L = 2, 16, 16
NW = NC * NS
C = 128


def _sc_deg(col_r, w_r, n_pad):
    T = col_r.shape[1]
    rows_per_tile = n_pad // NS
    mesh = plsc.VectorSubcoreMesh(core_axis_name="c", subcore_axis_name="s")

    @functools.partial(
        pl.kernel,
        mesh=mesh,
        out_type=jax.ShapeDtypeStruct((NC, n_pad), jnp.float32),
        scratch_types=[
            pltpu.VMEM((T, C), jnp.int32),
            pltpu.VMEM((T, C), jnp.float32),
            pltpu.VMEM((rows_per_tile,), jnp.float32),
            pltpu.VMEM_SHARED((n_pad,), jnp.float32),
        ],
    )
    def k(col_hbm, w_hbm, out_hbm, col_v, w_v, zbuf, acc_sh):
        c = lax.axis_index("c")
        s = lax.axis_index("s")
        wid = s * NC + c
        pltpu.sync_copy(col_hbm.at[wid], col_v)
        pltpu.sync_copy(w_hbm.at[wid], w_v)

        @pl.loop(0, rows_per_tile // L)
        def _zero(i):
            zbuf[pl.ds(i * L, L)] = jnp.zeros((L,), jnp.float32)

        pltpu.sync_copy(zbuf, acc_sh.at[pl.ds(s * rows_per_tile, rows_per_tile)])
        plsc.subcore_barrier()

        @pl.loop(0, T)
        def _chunk(j):
            pltpu.sync_copy(w_v.at[j], acc_sh.at[col_v.at[j]], add=True)

        plsc.subcore_barrier()
        pltpu.sync_copy(
            acc_sh.at[pl.ds(s * rows_per_tile, rows_per_tile)],
            out_hbm.at[c, pl.ds(s * rows_per_tile, rows_per_tile)],
        )

    return k(col_r, w_r)


def _sc_agg(hd, row_r, col_r, wf_r, d_pad, n_pad):
    T = row_r.shape[1]
    rows_per_tile = n_pad // NS
    zrows = rows_per_tile // 5
    mesh = plsc.VectorSubcoreMesh(core_axis_name="c", subcore_axis_name="s")

    @functools.partial(
        pl.kernel,
        mesh=mesh,
        out_type=jax.ShapeDtypeStruct((NC, n_pad, d_pad), jnp.float32),
        scratch_types=[
            pltpu.VMEM((T, C), jnp.int32),
            pltpu.VMEM((T, C), jnp.int32),
            pltpu.VMEM((T * C,), jnp.float32),
            pltpu.VMEM((C, d_pad), jnp.float32),
            pltpu.VMEM((zrows, d_pad), jnp.float32),
            pltpu.VMEM_SHARED((n_pad, d_pad), jnp.float32),
            pltpu.SemaphoreType.DMA,
        ],
        compiler_params=pltpu.CompilerParams(
            needs_layout_passes=False, use_tc_tiling_on_sc=False
        ),
    )
    def k(hd_hbm, row_hbm, col_hbm, w_hbm, out_hbm,
          row_v, col_v, w_v, msg_v, zbuf, acc_sh, sem):
        c = lax.axis_index("c")
        s = lax.axis_index("s")
        wid = s * NC + c
        pltpu.sync_copy(row_hbm.at[wid], row_v)
        pltpu.sync_copy(col_hbm.at[wid], col_v)
        pltpu.sync_copy(w_hbm.at[wid], w_v)

        @pl.loop(0, zrows)
        def _zero(r):
            for kk in range(d_pad // L):
                zbuf[r, pl.ds(kk * L, L)] = jnp.zeros((L,), jnp.float32)

        @pl.loop(0, 5)
        def _zcopy(kz):
            pltpu.sync_copy(
                zbuf, acc_sh.at[pl.ds(s * rows_per_tile + kz * zrows, zrows)]
            )

        plsc.subcore_barrier()

        @pl.loop(0, T)
        def _chunk(j):
            pltpu.async_copy(hd_hbm.at[row_v.at[j]], msg_v, sem).wait()

            @pl.loop(0, C)
            def _scale(e):
                idx = jnp.full((L,), j * C + e, jnp.int32)
                wv = plsc.load_gather(w_v, [idx])
                for kk in range(d_pad // L):
                    sl = pl.ds(kk * L, L)
                    msg_v[e, sl] = msg_v[e, sl] * wv

            pltpu.sync_copy(msg_v, acc_sh.at[col_v.at[j]], add=True)

        plsc.subcore_barrier()
        pltpu.sync_copy(
            acc_sh.at[pl.ds(s * rows_per_tile, rows_per_tile)],
            out_hbm.at[c, pl.ds(s * rows_per_tile, rows_per_tile)],
        )

    return k(hd, row_r, col_r, wf_r)


def _pad_d(x, d_pad):
    d = x.shape[1]
    if d == d_pad:
        return x
    return jnp.pad(x, ((0, 0), (0, d_pad - d)))


def kernel(features, edges, weights, W1, b1, W2, b2, W3, b3, W4, b4):
    n = features.shape[0]
    e_cnt = edges.shape[1]
    row = edges[0].astype(jnp.int32)
    col = edges[1].astype(jnp.int32)
    w = weights.astype(jnp.float32)

    T = -(-e_cnt // (NW * C))
    e_pad = NW * T * C
    npad = e_pad - e_cnt
    if npad:
        fill = (jnp.arange(npad, dtype=jnp.int32) * 97) % n
        row = jnp.concatenate([row, fill])
        col = jnp.concatenate([col, fill])
        w = jnp.concatenate([w, jnp.zeros((npad,), jnp.float32)])
    row_r = row.reshape(NW, T, C)
    col_r = col.reshape(NW, T, C)
    w_r = w.reshape(NW, T, C)
    wf_r = w.reshape(NW, T * C)

    n_pad = -(-n // (NS * 40)) * (NS * 40)
    degp = _sc_deg(col_r, w_r, n_pad)
    deg = 1.0 + degp[0, :n] + degp[1, :n]
    dinv = lax.rsqrt(deg)
    dinv_c = dinv[:, None]
    dinv2_c = (dinv * dinv)[:, None]

    x = features
    layers = [(W1, b1), (W2, b2), (W2, b2), (W3, b3), (W4, b4)]
    for i, (W, b) in enumerate(layers):
        d = W.shape[1]
        d_pad = -(-d // L) * L
        h = x @ W
        hd = _pad_d(h * dinv_c, d_pad)
        P = _sc_agg(hd, row_r, col_r, wf_r, d_pad, n_pad)
        S = (P[0, :n] + P[1, :n])[:, :d]
        out = dinv_c * S + dinv2_c * h + b
        if i < len(layers) - 1:
            x = jax.nn.relu(out)
        else:
            x = jax.nn.log_softmax(out, axis=1)
    return x


# speedup vs baseline: 54.5814x; 2.8108x over previous
import functools

import jax
import jax.numpy as jnp
from jax import lax
from jax.experimental import pallas as pl
from jax.experimental.pallas import tpu as pltpu, tpu_sc as plsc

NC, NS, L = 2, 16, 16
NW = NC * NS
C = 128
K = 4

_SC_PARAMS = pltpu.CompilerParams(
    needs_layout_passes=False, use_tc_tiling_on_sc=False
)


def _sc_deg(col_r, w_r, n_pad):
    T = col_r.shape[1]
    rows_per_tile = n_pad // NS
    mesh = plsc.VectorSubcoreMesh(core_axis_name="c", subcore_axis_name="s")

    @functools.partial(
        pl.kernel,
        mesh=mesh,
        out_type=jax.ShapeDtypeStruct((NC, n_pad), jnp.float32),
        scratch_types=[
            pltpu.VMEM((T, C), jnp.int32),
            pltpu.VMEM((T, C), jnp.float32),
            pltpu.VMEM((rows_per_tile,), jnp.float32),
            pltpu.VMEM_SHARED((n_pad,), jnp.float32),
            pltpu.SemaphoreType.DMA,
        ],
        compiler_params=_SC_PARAMS,
    )
    def k(col_hbm, w_hbm, out_hbm, col_v, w_v, zbuf, acc_sh, sem):
        c = lax.axis_index("c")
        s = lax.axis_index("s")
        wid = s * NC + c
        pltpu.sync_copy(col_hbm.at[wid], col_v)
        pltpu.sync_copy(w_hbm.at[wid], w_v)

        @pl.loop(0, rows_per_tile // L)
        def _zero(i):
            zbuf[pl.ds(i * L, L)] = jnp.zeros((L,), jnp.float32)

        pltpu.sync_copy(zbuf, acc_sh.at[pl.ds(s * rows_per_tile, rows_per_tile)])
        plsc.subcore_barrier()

        @pl.loop(0, T)
        def _fire(j):
            pltpu.async_copy(w_v.at[j], acc_sh.at[col_v.at[j]], sem, add=True)

        @pl.loop(0, T)
        def _drain(j):
            pltpu.make_async_copy(w_v.at[j], acc_sh.at[col_v.at[j]], sem).wait()

        plsc.subcore_barrier()
        pltpu.sync_copy(
            acc_sh.at[pl.ds(s * rows_per_tile, rows_per_tile)],
            out_hbm.at[c, pl.ds(s * rows_per_tile, rows_per_tile)],
        )

    return k(col_r, w_r)


def _sc_agg(hd, row_r, col_r, w_r, d_pad, n_pad):
    T = row_r.shape[1]
    rows_per_tile = n_pad // NS
    nz = rows_per_tile // C
    mesh = plsc.VectorSubcoreMesh(core_axis_name="c", subcore_axis_name="s")

    @functools.partial(
        pl.kernel,
        mesh=mesh,
        out_type=jax.ShapeDtypeStruct((NC, n_pad, d_pad), jnp.float32),
        scratch_types=[
            pltpu.VMEM((T, C), jnp.int32),
            pltpu.VMEM((T, C), jnp.int32),
            pltpu.VMEM((T, C), jnp.float32),
            pltpu.VMEM((K, C, d_pad), jnp.float32),
            pltpu.VMEM_SHARED((n_pad, d_pad), jnp.float32),
        ] + [pltpu.SemaphoreType.DMA] * K,
        compiler_params=_SC_PARAMS,
    )
    def k(hd_hbm, row_hbm, col_hbm, w_hbm, out_hbm,
          row_v, col_v, w_v, msg_v, acc_sh, *sems):
        c = lax.axis_index("c")
        s = lax.axis_index("s")
        wid = s * NC + c
        pltpu.sync_copy(row_hbm.at[wid], row_v)
        pltpu.sync_copy(col_hbm.at[wid], col_v)
        pltpu.sync_copy(w_hbm.at[wid], w_v)

        zb = msg_v.at[0]

        @pl.loop(0, (C * d_pad) // L)
        def _zero(i):
            r = i // (d_pad // L)
            kk = i % (d_pad // L)
            zb[r, pl.ds(kk * L, L)] = jnp.zeros((L,), jnp.float32)

        @pl.loop(0, nz)
        def _zcopy(kz):
            pltpu.sync_copy(zb, acc_sh.at[pl.ds(s * rows_per_tile + kz * C, C)])

        plsc.subcore_barrier()

        for b in range(K):
            pltpu.async_copy(hd_hbm.at[row_v.at[b]], msg_v.at[b], sems[b])

        def scale_chunk(mb, j):
            @pl.loop(0, C // 16)
            def _grp(q):
                wrow = w_v[j, pl.ds(q * 16, 16)]
                for l in range(16):
                    wv = jnp.full((L,), wrow[l], jnp.float32)
                    e = q * 16 + l
                    for kk in range(d_pad // L):
                        sl = pl.ds(kk * L, L)
                        mb[e, sl] = mb[e, sl] * wv

        @pl.loop(0, T // K)
        def _ring(gi):
            for b in range(K):
                j = gi * K + b
                mb = msg_v.at[b]
                pltpu.make_async_copy(
                    hd_hbm.at[row_v.at[j]], mb, sems[b]
                ).wait()
                scale_chunk(mb, j)
                pltpu.sync_copy(mb, acc_sh.at[col_v.at[j]], add=True)

                @pl.when(j + K < T)
                def _refill():
                    pltpu.async_copy(
                        hd_hbm.at[row_v.at[j + K]], mb, sems[b]
                    )

        plsc.subcore_barrier()
        pltpu.sync_copy(
            acc_sh.at[pl.ds(s * rows_per_tile, rows_per_tile)],
            out_hbm.at[c, pl.ds(s * rows_per_tile, rows_per_tile)],
        )

    return k(hd, row_r, col_r, w_r)


def _pad_d(x, d_pad):
    d = x.shape[1]
    if d == d_pad:
        return x
    return jnp.pad(x, ((0, 0), (0, d_pad - d)))


def kernel(features, edges, weights, W1, b1, W2, b2, W3, b3, W4, b4):
    n = features.shape[0]
    e_cnt = edges.shape[1]
    row = edges[0].astype(jnp.int32)
    col = edges[1].astype(jnp.int32)
    w = weights.astype(jnp.float32)

    T = -(-e_cnt // (NW * C * K)) * K
    e_pad = NW * T * C
    npad = e_pad - e_cnt
    if npad:
        fill = (jnp.arange(npad, dtype=jnp.int32) * 97) % n
        row = jnp.concatenate([row, fill])
        col = jnp.concatenate([col, fill])
        w = jnp.concatenate([w, jnp.zeros((npad,), jnp.float32)])
    row_r = row.reshape(NW, T, C)
    col_r = col.reshape(NW, T, C)
    w_r = w.reshape(NW, T, C)

    n_pad = -(-n // (NS * C)) * (NS * C)
    degp = _sc_deg(col_r, w_r, n_pad)
    deg = 1.0 + degp[0, :n] + degp[1, :n]
    dinv = lax.rsqrt(deg)
    dinv_c = dinv[:, None]
    dinv2_c = (dinv * dinv)[:, None]

    x = features
    layers = [(W1, b1), (W2, b2), (W2, b2), (W3, b3), (W4, b4)]
    for i, (W, b) in enumerate(layers):
        d = W.shape[1]
        d_pad = -(-d // L) * L
        h = x @ W
        hd = _pad_d(h * dinv_c, d_pad)
        P = _sc_agg(hd, row_r, col_r, w_r, d_pad, n_pad)
        S = (P[0, :n] + P[1, :n])[:, :d]
        out = dinv_c * S + dinv2_c * h + b
        if i < len(layers) - 1:
            x = jax.nn.relu(out)
        else:
            x = jax.nn.log_softmax(out, axis=1)
    return x


# speedup vs baseline: 67.9209x; 1.2444x over previous
import functools

import jax
import jax.numpy as jnp
from jax import lax
from jax.experimental import pallas as pl
from jax.experimental.pallas import tpu as pltpu, tpu_sc as plsc

NC, NS, L = 2, 16, 16
NW = NC * NS
C = 128
K = 4

_SC_PARAMS = pltpu.CompilerParams(
    needs_layout_passes=False, use_tc_tiling_on_sc=False
)


def _sc_deg(col_r, w_r, n_pad):
    T = col_r.shape[1]
    rows_per_tile = n_pad // NS
    mesh = plsc.VectorSubcoreMesh(core_axis_name="c", subcore_axis_name="s")

    @functools.partial(
        pl.kernel,
        mesh=mesh,
        out_type=jax.ShapeDtypeStruct((NC, n_pad), jnp.float32),
        scratch_types=[
            pltpu.VMEM((T, C), jnp.int32),
            pltpu.VMEM((T, C), jnp.float32),
            pltpu.VMEM((rows_per_tile,), jnp.float32),
            pltpu.VMEM_SHARED((n_pad,), jnp.float32),
            pltpu.SemaphoreType.DMA,
        ],
        compiler_params=_SC_PARAMS,
    )
    def k(col_hbm, w_hbm, out_hbm, col_v, w_v, zbuf, acc_sh, sem):
        c = lax.axis_index("c")
        s = lax.axis_index("s")
        wid = s * NC + c
        pltpu.sync_copy(col_hbm.at[wid], col_v)
        pltpu.sync_copy(w_hbm.at[wid], w_v)

        @pl.loop(0, rows_per_tile // L)
        def _zero(i):
            zbuf[pl.ds(i * L, L)] = jnp.zeros((L,), jnp.float32)

        pltpu.sync_copy(zbuf, acc_sh.at[pl.ds(s * rows_per_tile, rows_per_tile)])
        plsc.subcore_barrier()

        @pl.loop(0, T)
        def _fire(j):
            pltpu.async_copy(w_v.at[j], acc_sh.at[col_v.at[j]], sem, add=True)

        @pl.loop(0, T)
        def _drain(j):
            pltpu.make_async_copy(w_v.at[j], acc_sh.at[col_v.at[j]], sem).wait()

        plsc.subcore_barrier()
        pltpu.sync_copy(
            acc_sh.at[pl.ds(s * rows_per_tile, rows_per_tile)],
            out_hbm.at[c, pl.ds(s * rows_per_tile, rows_per_tile)],
        )

    return k(col_r, w_r)


def _sc_agg(hd, row_r, col_r, w_r, d_pad, n_pad):
    T = row_r.shape[1]
    rows_per_tile = n_pad // NS
    nz = rows_per_tile // C
    R = 2 * K
    mesh = plsc.VectorSubcoreMesh(core_axis_name="c", subcore_axis_name="s")

    @functools.partial(
        pl.kernel,
        mesh=mesh,
        out_type=jax.ShapeDtypeStruct((NC, n_pad, d_pad), jnp.float32),
        scratch_types=[
            pltpu.VMEM((T, C), jnp.int32),
            pltpu.VMEM((T, C), jnp.int32),
            pltpu.VMEM((T, C), jnp.float32),
            pltpu.VMEM((R, C, d_pad), jnp.float32),
            pltpu.VMEM_SHARED((n_pad, d_pad), jnp.float32),
        ] + [pltpu.SemaphoreType.DMA] * (2 * R),
        compiler_params=_SC_PARAMS,
    )
    def k(hd_hbm, row_hbm, col_hbm, w_hbm, out_hbm,
          row_v, col_v, w_v, msg_v, acc_sh, *sems):
        gsem = sems[:R]
        ssem = sems[R:]
        c = lax.axis_index("c")
        s = lax.axis_index("s")
        wid = s * NC + c
        pltpu.sync_copy(row_hbm.at[wid], row_v)
        pltpu.sync_copy(col_hbm.at[wid], col_v)
        pltpu.sync_copy(w_hbm.at[wid], w_v)

        zb = msg_v.at[0]

        @pl.loop(0, (C * d_pad) // L)
        def _zero(i):
            r = i // (d_pad // L)
            kk = i % (d_pad // L)
            zb[r, pl.ds(kk * L, L)] = jnp.zeros((L,), jnp.float32)

        @pl.loop(0, nz)
        def _zcopy(kz):
            pltpu.sync_copy(zb, acc_sh.at[pl.ds(s * rows_per_tile + kz * C, C)])

        plsc.subcore_barrier()

        for b in range(K):
            pltpu.async_copy(hd_hbm.at[row_v.at[b]], msg_v.at[b], gsem[b])

        def scale_chunk(mb, j):
            @pl.loop(0, C // 16)
            def _grp(q):
                wrow = w_v[j, pl.ds(q * 16, 16)]
                for l in range(16):
                    wv = jnp.full((L,), wrow[l], jnp.float32)
                    e = q * 16 + l
                    for kk in range(d_pad // L):
                        sl = pl.ds(kk * L, L)
                        mb[e, sl] = mb[e, sl] * wv

        @pl.loop(0, T // R)
        def _ring(gi):
            for v in range(R):
                j = gi * R + v
                mb = msg_v.at[v]
                pltpu.make_async_copy(hd_hbm.at[row_v.at[j]], mb, gsem[v]).wait()
                scale_chunk(mb, j)
                pltpu.async_copy(mb, acc_sh.at[col_v.at[j]], ssem[v], add=True)

                jg = j + K
                bg = (v + K) % R
                mg = msg_v.at[bg]

                @pl.when(jg < T)
                def _refill():
                    @pl.when(jg >= R)
                    def _drain_prev_scatter():
                        pltpu.make_async_copy(
                            mg, acc_sh.at[col_v.at[jg - R]], ssem[bg]
                        ).wait()

                    pltpu.async_copy(hd_hbm.at[row_v.at[jg]], mg, gsem[bg])

        for b in range(R):
            pltpu.make_async_copy(
                msg_v.at[b], acc_sh.at[col_v.at[T - R + b]], ssem[b]
            ).wait()

        plsc.subcore_barrier()
        pltpu.sync_copy(
            acc_sh.at[pl.ds(s * rows_per_tile, rows_per_tile)],
            out_hbm.at[c, pl.ds(s * rows_per_tile, rows_per_tile)],
        )

    return k(hd, row_r, col_r, w_r)


def _pad_d(x, d_pad):
    d = x.shape[1]
    if d == d_pad:
        return x
    return jnp.pad(x, ((0, 0), (0, d_pad - d)))


def kernel(features, edges, weights, W1, b1, W2, b2, W3, b3, W4, b4):
    n = features.shape[0]
    e_cnt = edges.shape[1]
    row = edges[0].astype(jnp.int32)
    col = edges[1].astype(jnp.int32)
    w = weights.astype(jnp.float32)

    T = -(-e_cnt // (NW * C * 2 * K)) * (2 * K)
    e_pad = NW * T * C
    npad = e_pad - e_cnt
    if npad:
        fill = (jnp.arange(npad, dtype=jnp.int32) * 97) % n
        row = jnp.concatenate([row, fill])
        col = jnp.concatenate([col, fill])
        w = jnp.concatenate([w, jnp.zeros((npad,), jnp.float32)])
    row_r = row.reshape(NW, T, C)
    col_r = col.reshape(NW, T, C)
    w_r = w.reshape(NW, T, C)

    n_pad = -(-n // (NS * C)) * (NS * C)
    degp = _sc_deg(col_r, w_r, n_pad)
    deg = 1.0 + degp[0, :n] + degp[1, :n]
    dinv = lax.rsqrt(deg)
    dinv_c = dinv[:, None]
    dinv2_c = (dinv * dinv)[:, None]

    x = features
    layers = [(W1, b1), (W2, b2), (W2, b2), (W3, b3), (W4, b4)]
    for i, (W, b) in enumerate(layers):
        d_in, d_out = W.shape
        if d_out <= d_in:
            d_pad = -(-d_out // L) * L
            h = x @ W
            hd = _pad_d(h * dinv_c, d_pad)
            P = _sc_agg(hd, row_r, col_r, w_r, d_pad, n_pad)
            S = (P[0, :n] + P[1, :n])[:, :d_out]
            out = dinv_c * S + dinv2_c * h + b
        else:
            d_pad = -(-d_in // L) * L
            xd = _pad_d(x * dinv_c, d_pad)
            P = _sc_agg(xd, row_r, col_r, w_r, d_pad, n_pad)
            S = (P[0, :n] + P[1, :n])[:, :d_in]
            ax = dinv_c * S + dinv2_c * x
            out = ax @ W + b
        if i < len(layers) - 1:
            x = jax.nn.relu(out)
        else:
            x = jax.nn.log_softmax(out, axis=1)
    return x
